# baseline (speedup 1.0000x reference)
import functools

import jax
import jax.numpy as jnp
from jax import lax
from jax.experimental import pallas as pl
from jax.experimental.pallas import tpu as pltpu
from jax.experimental.pallas import tpu_sc as plsc

VOCAB = 1_000_000
DIM = 64
SEQ = 50
NROW = 16384

NC = 2
NS = 16
NW = NC * NS

NPW = NROW // NW
NL = 256
NB = NPW // NL
K = SEQ * NB
NBUF = 4
OUTER = K // NBUF

_mesh = plsc.VectorSubcoreMesh(
    core_axis_name="c", subcore_axis_name="s", num_cores=NC, num_subcores=NS
)


@functools.partial(
    pl.kernel,
    out_type=jax.ShapeDtypeStruct((SEQ * NROW, DIM), jnp.float32),
    mesh=_mesh,
    scratch_types=[
        pltpu.VMEM((SEQ, NPW), jnp.int32),
        pltpu.VMEM((NBUF, NL, DIM), jnp.float32),
        [pltpu.SemaphoreType.DMA] * NBUF,
    ],
    compiler_params=pltpu.CompilerParams(use_tc_tiling_on_sc=False),
)
def _emb_gather(xt_hbm, table_hbm, out_hbm, idx_v, rows_v, gsems):
    wid = lax.axis_index("s") * NC + lax.axis_index("c")
    nbase = wid * NPW
    pltpu.sync_copy(xt_hbm.at[:, pl.ds(nbase, NPW)], idx_v)

    def fire(kk, b):
        s = kk // NB
        nb = kk % NB
        pltpu.async_copy(table_hbm.at[idx_v.at[s, pl.ds(nb * NL, NL)]],
                         rows_v.at[b], gsems[b])

    def drain(kk, b):
        s = kk // NB
        nb = kk % NB
        pltpu.make_async_copy(table_hbm.at[idx_v.at[s, pl.ds(nb * NL, NL)]],
                              rows_v.at[b], gsems[b]).wait()
        pltpu.sync_copy(rows_v.at[b],
                        out_hbm.at[pl.ds(s * NROW + nbase + nb * NL, NL)])

    for b in range(NBUF):
        fire(b, b)

    @pl.loop(0, OUTER - 1)
    def _outer(o):
        for b in range(NBUF):
            kk = o * NBUF + b
            drain(kk, b)
            fire(kk + NBUF, b)

    for b in range(NBUF):
        drain((OUTER - 1) * NBUF + b, b)


_TN = 512


def _fmt_body(g_ref, o_ref):
    o_ref[0] = g_ref[...].T


_tc_format = pl.pallas_call(
    _fmt_body,
    grid=(SEQ, NROW // _TN),
    in_specs=[pl.BlockSpec((_TN, DIM), lambda s, nb: (s * (NROW // _TN) + nb, 0))],
    out_specs=pl.BlockSpec((1, DIM, _TN), lambda s, nb: (s, 0, nb)),
    out_shape=jax.ShapeDtypeStruct((SEQ, DIM, NROW), jnp.float32),
)


def kernel(x, emb):
    g2 = _emb_gather(x.T, emb)
    ol = _tc_format(g2)
    return ol.transpose(2, 0, 1)
